# baseline (speedup 1.0000x reference)
import functools

import jax
import jax.numpy as jnp
from jax import lax
from jax.experimental import pallas as pl
from jax.experimental.pallas import tpu as pltpu
from jax.experimental.pallas import tpu_sc as plsc

NUM_SCENES = 1000000
LATENT = 64
BATCH = 16384
LANES = 16
NUM_CORES = 2
NUM_SUBCORES = 16
NUM_WORKERS = NUM_CORES * NUM_SUBCORES
PAGE = 128
NUM_PAGES = (NUM_SCENES + PAGE - 1) // PAGE
PPW = (NUM_PAGES + NUM_WORKERS - 1) // NUM_WORKERS
SLOTS = 8
NBUF = 2
VECS_PER_ROW = LATENT // LANES
NCHUNKS = BATCH // LANES

_GATHER_DNUMS = lax.GatherDimensionNumbers(
    offset_dims=(), collapsed_slice_dims=(0,), start_index_map=(0,)
)


def _permute(v, idx):
    return lax.gather(
        v,
        idx[:, None],
        _GATHER_DNUMS,
        (1,),
        mode=lax.GatherScatterMode.PROMISE_IN_BOUNDS,
    )


def _rsqrt(x):
    i = lax.bitcast_convert_type(x, jnp.int32)
    i = jnp.int32(0x5F3759DF) - lax.shift_right_arithmetic(i, 1)
    y = lax.bitcast_convert_type(i, jnp.float32)
    for _ in range(3):
        y = y * (1.5 - 0.5 * x * y * y)
    return y


@functools.partial(
    pl.kernel,
    out_type=jax.ShapeDtypeStruct((BATCH, LATENT), jnp.float32),
    mesh=plsc.VectorSubcoreMesh(core_axis_name="c", subcore_axis_name="s"),
    scratch_types=[
        pltpu.VMEM((BATCH,), jnp.int32),
        pltpu.VMEM((BATCH,), jnp.int32),
        pltpu.VMEM((BATCH,), jnp.int32),
        pltpu.VMEM((PPW * SLOTS * LANES,), jnp.int32),
        pltpu.VMEM((NBUF, LATENT, PAGE), jnp.float32),
        pltpu.VMEM((NBUF, SLOTS, LATENT), jnp.float32),
        pltpu.SMEM((PPW,), jnp.int32),
        pltpu.SMEM((PPW,), jnp.int32),
        pltpu.SemaphoreType.DMA,
        pltpu.SemaphoreType.DMA,
        pltpu.SemaphoreType.DMA,
    ],
)
def _gather_maxnorm(
    idx_hbm,
    tab_t_hbm,
    out_hbm,
    idx_all,
    stage_v,
    flag_v,
    slot_v,
    page_v,
    row_v,
    cnt_s,
    plist_s,
    sem0,
    sem1,
    sem_out,
):
    wid = lax.axis_index("s") * NUM_CORES + lax.axis_index("c")
    p_base = wid * PPW
    page_sems = [sem0, sem1]

    pltpu.sync_copy(idx_hbm, idx_all)

    lanes = lax.iota(jnp.int32, LANES)
    perms = [lanes ^ sh for sh in (8, 4, 2, 1)]
    lane_eq = [lanes == l for l in range(LANES)]
    zero_f = jnp.zeros((LANES,), jnp.float32)

    def page_fetch(lp, buf, sem):
        off = pl.multiple_of((p_base + lp) * PAGE, PAGE)
        return pltpu.async_copy(
            tab_t_hbm.at[:, pl.ds(off, PAGE)], page_v.at[buf], sem
        )

    def process_entry(e, buf, k):
        col = lax.shift_right_logical(e, 14) & 127
        m = e & (BATCH - 1)
        g0 = col & ~15
        lv = lanes * 0 + (col & 15)
        acc = zero_f
        rows = [zero_f] * VECS_PER_ROW
        for q in range(VECS_PER_ROW):
            r = rows[q]
            for l in range(LANES):
                j = q * LANES + l
                vj = page_v[buf, j, pl.ds(g0, LANES)]
                acc = acc + vj * vj
                r = jnp.where(lane_eq[l], _permute(vj, lv), r)
            rows[q] = r
        scale = jnp.minimum(1.0, _rsqrt(_permute(acc, lv)))
        for q in range(VECS_PER_ROW):
            row_v[buf, k, pl.ds(q * LANES, LANES)] = rows[q] * scale
        pltpu.async_copy(
            row_v.at[buf].at[pl.ds(k, 1), :],
            out_hbm.at[pl.ds(m, 1), :],
            sem_out,
        )

    def drain_out():
        pltpu.make_async_copy(
            row_v.at[0].at[pl.ds(0, 1), :],
            out_hbm.at[pl.ds(0, 1), :],
            sem_out,
        ).wait()

    def scan_body(g, carry):
        v = idx_all[pl.ds(g * LANES, LANES)]
        pg = lax.shift_right_logical(v, 7)
        lp = pg - p_base
        mine = jnp.logical_and(lp >= 0, lp < PPW)
        m = g * LANES + lanes
        entry = (
            lax.shift_left(lp, 21)
            | lax.shift_left(v & 127, 14)
            | m
        )
        stage_v[pl.ds(g * LANES, LANES)] = jnp.where(mine, entry, jnp.int32(-1))
        anyv = jnp.where(mine, jnp.int32(1), jnp.int32(0))
        for p in perms:
            anyv = jnp.maximum(anyv, _permute(anyv, p))
        flag_v[pl.ds(g * LANES, LANES)] = anyv
        return carry

    lax.fori_loop(0, NCHUNKS, scan_body, 0)

    def zero_body(p, carry):
        cnt_s[p] = 0
        return carry

    lax.fori_loop(0, PPW, zero_body, 0)

    def bucket_chunk(t, carry):
        anyhit = flag_v[pl.ds(t * LANES, LANES)][0]

        @pl.when(anyhit > 0)
        def _():
            evec = stage_v[pl.ds(t * LANES, LANES)]
            for l in range(LANES):
                e = evec[l]

                @pl.when(e >= 0)
                def _():
                    lp = lax.shift_right_logical(e, 21)
                    c = cnt_s[lp]
                    cnt_s[lp] = c + 1

                    @pl.when(c < SLOTS)
                    def _():
                        slot_v[pl.ds((lp * SLOTS + c) * LANES, LANES)] = (
                            lanes * 0 + e
                        )

                    @pl.when(c >= SLOTS)
                    def _():
                        page_fetch(lp, 0, sem0).wait()
                        process_entry(e, 0, 0)
                        drain_out()

        return carry

    lax.fori_loop(0, NCHUNKS, bucket_chunk, 0)

    def plist_body(p, np_):
        c = cnt_s[p]

        @pl.when(c > 0)
        def _():
            plist_s[np_] = p

        return jnp.where(c > 0, np_ + 1, np_)

    npages = lax.fori_loop(0, PPW, plist_body, jnp.int32(0))

    for b in range(NBUF):

        @pl.when(b < npages)
        def _():
            page_fetch(plist_s[b], b, page_sems[b])

    def ring_body(g2, carry):
        for b in range(NBUF):
            gidx = g2 * NBUF + b

            @pl.when(gidx < npages)
            def _():
                lp = plist_s[gidx]
                c = cnt_s[lp]
                pltpu.make_async_copy(
                    tab_t_hbm.at[:, pl.ds(0, PAGE)],
                    page_v.at[b],
                    page_sems[b],
                ).wait()
                nin = jnp.minimum(c, SLOTS)
                for k in range(SLOTS):

                    @pl.when(k < nin)
                    def _():
                        ev = slot_v[pl.ds((lp * SLOTS + k) * LANES, LANES)]
                        process_entry(ev[0], b, k)

                for k in range(SLOTS):

                    @pl.when(k < nin)
                    def _():
                        drain_out()

                nxt = gidx + NBUF

                @pl.when(nxt < npages)
                def _():
                    page_fetch(plist_s[nxt], b, page_sems[b])

        return carry

    nrounds = lax.div(npages + (NBUF - 1), jnp.int32(NBUF))
    lax.fori_loop(0, nrounds, ring_body, 0)


def kernel(idxs, table):
    return _gather_maxnorm(idxs.astype(jnp.int32), jnp.transpose(table))


# speedup vs baseline: 2.9384x; 2.9384x over previous
import functools

import jax
import jax.numpy as jnp
from jax import lax
from jax.experimental import pallas as pl
from jax.experimental.pallas import tpu as pltpu
from jax.experimental.pallas import tpu_sc as plsc

NUM_SCENES = 1000000
LATENT = 64
BATCH = 16384
LANES = 16
NUM_CORES = 2
NUM_SUBCORES = 16
NUM_WORKERS = NUM_CORES * NUM_SUBCORES
BPW = BATCH // NUM_WORKERS
VECS_PER_ROW = LATENT // LANES
CHUNK = 16

_GATHER_DNUMS = lax.GatherDimensionNumbers(
    offset_dims=(), collapsed_slice_dims=(0,), start_index_map=(0,)
)


def _permute(v, idx):
    return lax.gather(
        v,
        idx[:, None],
        _GATHER_DNUMS,
        (1,),
        mode=lax.GatherScatterMode.PROMISE_IN_BOUNDS,
    )


def _rsqrt(x):
    i = lax.bitcast_convert_type(x, jnp.int32)
    i = jnp.int32(0x5F3759DF) - lax.shift_right_arithmetic(i, 1)
    y = lax.bitcast_convert_type(i, jnp.float32)
    for _ in range(3):
        y = y * (1.5 - 0.5 * x * y * y)
    return y


@functools.partial(
    pl.kernel,
    out_type=jax.ShapeDtypeStruct((BATCH, LATENT), jnp.float32),
    mesh=plsc.VectorSubcoreMesh(core_axis_name="c", subcore_axis_name="s"),
    scratch_types=[
        pltpu.VMEM((BPW,), jnp.int32),
        pltpu.VMEM((BPW, LATENT), jnp.float32),
        pltpu.SemaphoreType.DMA,
    ],
)
def _gather_maxnorm(idx_hbm, table_hbm, out_hbm, idx_v, rows_v, sem):
    wid = lax.axis_index("s") * NUM_CORES + lax.axis_index("c")
    base = wid * BPW
    pltpu.sync_copy(idx_hbm.at[pl.ds(base, BPW)], idx_v)

    def gather_chunk(c, carry):
        r0 = c * CHUNK
        ivec = idx_v[pl.ds(r0, CHUNK)]
        cps = []
        for j in range(CHUNK):
            i = ivec[j]
            cps.append(
                pltpu.async_copy(
                    table_hbm.at[pl.ds(i, 1), :],
                    rows_v.at[pl.ds(r0 + j, 1), :],
                    sem,
                )
            )
        for cp in cps:
            cp.wait()
        return carry

    lax.fori_loop(0, BPW // CHUNK, gather_chunk, 0)

    lanes = lax.iota(jnp.int32, LANES)
    perms = [lanes ^ sh for sh in (8, 4, 2, 1)]

    def row_fn(r, carry):
        vecs = [rows_v[r, pl.ds(j * LANES, LANES)] for j in range(VECS_PER_ROW)]
        acc = vecs[0] * vecs[0]
        for v in vecs[1:]:
            acc = acc + v * v
        for p in perms:
            acc = acc + _permute(acc, p)
        scale = jnp.minimum(1.0, _rsqrt(acc))
        for j in range(VECS_PER_ROW):
            rows_v[r, pl.ds(j * LANES, LANES)] = vecs[j] * scale
        return carry

    lax.fori_loop(0, BPW, row_fn, 0)
    pltpu.sync_copy(rows_v, out_hbm.at[pl.ds(base, BPW), :])


def kernel(idxs, table):
    return _gather_maxnorm(idxs.astype(jnp.int32), table)
